# baseline (speedup 1.0000x reference)
import functools

import jax
import jax.numpy as jnp
from jax import lax
from jax.experimental import pallas as pl
from jax.experimental.pallas import tpu as pltpu
from jax.experimental.pallas import tpu_sc as plsc

_NC = 2
_NS = 16
_NW = _NC * _NS

_CB = 128
_NBUF = 5


def _sc_gather(table, idx_t, b, f, d):
    per_w = b // _NW
    n_cb = per_w // _CB
    n_chunks = f * n_cb
    n_groups = n_chunks // _NBUF

    mesh = plsc.VectorSubcoreMesh(core_axis_name="c", subcore_axis_name="s")

    @functools.partial(
        pl.kernel,
        out_type=jax.ShapeDtypeStruct((f, b, d), jnp.float32),
        mesh=mesh,
        compiler_params=pltpu.CompilerParams(use_tc_tiling_on_sc=True),
        scratch_types=[
            pltpu.VMEM((f, per_w), jnp.int32),
            pltpu.VMEM((_NBUF * _CB, d), jnp.float32),
            pltpu.VMEM_SHARED((129, d), jnp.float32),
        ] + [pltpu.SemaphoreType.DMA] * (2 * _NBUF),
    )
    def k(table_hbm, idx_hbm, out_hbm, idx_v, rows_v, table_sh, *sems):
        gsem = sems[:_NBUF]
        ssem = sems[_NBUF:]
        sid = lax.axis_index("s")
        wid = sid * _NC + lax.axis_index("c")
        col_base = wid * per_w

        @pl.when(sid == 0)
        def _():
            pltpu.sync_copy(table_hbm, table_sh)

        pltpu.sync_copy(idx_hbm.at[:, pl.ds(col_base, per_w)], idx_v)
        plsc.subcore_barrier()

        def store_wait(bf):
            pltpu.make_async_copy(
                rows_v.at[pl.ds(bf * _CB, _CB)],
                out_hbm.at[0, pl.ds(col_base, _CB)],
                ssem[bf],
            ).wait()

        def body(g, carry):
            t0 = g * _NBUF
            handles = []
            for bf in range(_NBUF):
                t = t0 + bf
                fi = t // n_cb
                cb = t % n_cb
                @pl.when(g > 0)
                def _(bf=bf):
                    store_wait(bf)

                handles.append((
                    fi,
                    cb,
                    pltpu.async_copy(
                        table_sh.at[idx_v.at[fi, pl.ds(cb * _CB, _CB)]],
                        rows_v.at[pl.ds(bf * _CB, _CB)],
                        gsem[bf],
                    ),
                ))
            for bf in range(_NBUF):
                fi, cb, h = handles[bf]
                h.wait()
                pltpu.async_copy(
                    rows_v.at[pl.ds(bf * _CB, _CB)],
                    out_hbm.at[fi, pl.ds(col_base + cb * _CB, _CB)],
                    ssem[bf],
                )
            return carry

        lax.fori_loop(0, n_groups, body, 0)
        for bf in range(_NBUF):
            store_wait(bf)

    return k(table, idx_t)


def kernel(inputs, embedding_matrix):
    b, f = inputs.shape
    d = embedding_matrix.shape[1]
    padded = jnp.concatenate(
        [jnp.zeros((1, d), embedding_matrix.dtype), embedding_matrix], axis=0
    )
    out_t = _sc_gather(padded, inputs.T.astype(jnp.int32), b, f, d)
    return jnp.transpose(out_t, (1, 0, 2))


# speedup vs baseline: 1.0178x; 1.0178x over previous
import functools

import jax
import jax.numpy as jnp
from jax import lax
from jax.experimental import pallas as pl
from jax.experimental.pallas import tpu as pltpu
from jax.experimental.pallas import tpu_sc as plsc

_NC = 2
_NS = 16
_NW = _NC * _NS

_CB = 64
_NBUF = 8


def _sc_gather(table, idx_t, b, f, d):
    per_w = b // _NW
    n_cb = per_w // _CB
    n_chunks = f * n_cb
    n_groups = n_chunks // _NBUF

    mesh = plsc.VectorSubcoreMesh(core_axis_name="c", subcore_axis_name="s")

    @functools.partial(
        pl.kernel,
        out_type=jax.ShapeDtypeStruct((f, b, d), jnp.float32),
        mesh=mesh,
        compiler_params=pltpu.CompilerParams(use_tc_tiling_on_sc=True),
        scratch_types=[
            pltpu.VMEM((f, per_w), jnp.int32),
            pltpu.VMEM((_NBUF * _CB, d), jnp.float32),
            pltpu.VMEM_SHARED((129, d), jnp.float32),
        ] + [pltpu.SemaphoreType.DMA] * (2 * _NBUF),
    )
    def k(table_hbm, idx_hbm, out_hbm, idx_v, rows_v, table_sh, *sems):
        gsem = sems[:_NBUF]
        ssem = sems[_NBUF:]
        sid = lax.axis_index("s")
        wid = sid * _NC + lax.axis_index("c")
        col_base = wid * per_w

        @pl.when(sid == 0)
        def _():
            pltpu.sync_copy(table_hbm, table_sh)

        pltpu.sync_copy(idx_hbm.at[:, pl.ds(col_base, per_w)], idx_v)
        plsc.subcore_barrier()

        def store_wait(bf):
            pltpu.make_async_copy(
                rows_v.at[pl.ds(bf * _CB, _CB)],
                out_hbm.at[0, pl.ds(col_base, _CB)],
                ssem[bf],
            ).wait()

        def body(g, carry):
            t0 = g * _NBUF
            handles = []
            for bf in range(_NBUF):
                t = t0 + bf
                fi = t // n_cb
                cb = t % n_cb
                @pl.when(g > 0)
                def _(bf=bf):
                    store_wait(bf)

                handles.append((
                    fi,
                    cb,
                    pltpu.async_copy(
                        table_sh.at[idx_v.at[fi, pl.ds(cb * _CB, _CB)]],
                        rows_v.at[pl.ds(bf * _CB, _CB)],
                        gsem[bf],
                    ),
                ))
            for bf in range(_NBUF):
                fi, cb, h = handles[bf]
                h.wait()
                pltpu.async_copy(
                    rows_v.at[pl.ds(bf * _CB, _CB)],
                    out_hbm.at[fi, pl.ds(col_base + cb * _CB, _CB)],
                    ssem[bf],
                )
            return carry

        lax.fori_loop(0, n_groups, body, 0)
        for bf in range(_NBUF):
            store_wait(bf)

    return k(table, idx_t)


def kernel(inputs, embedding_matrix):
    b, f = inputs.shape
    d = embedding_matrix.shape[1]
    padded = jnp.concatenate(
        [jnp.zeros((1, d), embedding_matrix.dtype), embedding_matrix], axis=0
    )
    out_t = _sc_gather(padded, inputs.T.astype(jnp.int32), b, f, d)
    return jnp.transpose(out_t, (1, 0, 2))
